# baseline (speedup 1.0000x reference)
import functools

import jax
import jax.numpy as jnp
import numpy as np
from jax import lax
from jax.experimental import pallas as pl
from jax.experimental.pallas import tpu as pltpu
from jax.experimental.pallas import tpu_sc as plsc

N_NODES = 10000
N_EDGES = 320000
H = 8
F = 40
DD = 16

NW = 32
CH = 128
K = 80
KH = 20
EW = K * CH
E_PAD = NW * EW
NC = NW * K
N_PAD = 10240
NBLK = 256
TW = 24
MB = 16
EBLK = MB * CH


def _prep_body(f_ref, s_ref, w1a_ref, w1b_ref, tsrc_ref, tdst_ref):
    f = f_ref[...]
    s = s_ref[...]
    a = jnp.dot(f, w1a_ref[...], preferred_element_type=jnp.float32)
    b = jnp.dot(f, w1b_ref[...], preferred_element_type=jnp.float32)
    tsrc_ref[...] = jnp.concatenate([a, s], axis=1)
    tdst_ref[...] = jnp.concatenate([b, s], axis=1)


def _prep(feature_p, state_p, w1a, w1b):
    grid = (N_PAD // NBLK,)
    return pl.pallas_call(
        _prep_body,
        grid=grid,
        in_specs=[
            pl.BlockSpec((NBLK, F), lambda i: (i, 0)),
            pl.BlockSpec((NBLK, H), lambda i: (i, 0)),
            pl.BlockSpec((F, 16), lambda i: (0, 0)),
            pl.BlockSpec((F, 16), lambda i: (0, 0)),
        ],
        out_specs=[
            pl.BlockSpec((NBLK, TW), lambda i: (i, 0)),
            pl.BlockSpec((NBLK, TW), lambda i: (i, 0)),
        ],
        out_shape=[
            jax.ShapeDtypeStruct((N_PAD, TW), jnp.float32),
            jax.ShapeDtypeStruct((N_PAD, TW), jnp.float32),
        ],
    )(feature_p, state_p, w1a, w1b)


def _transpose_chunk(rows_b, slab_b):
    def group(g, carry):
        e0 = g * 16
        eidx = lax.iota(jnp.int32, 16) + e0
        for t in range(TW):
            v = plsc.load_gather(rows_b, [eidx, jnp.full((16,), t, jnp.int32)])
            slab_b[t, pl.ds(e0, 16)] = v
        return carry

    lax.fori_loop(0, CH // 16, group, 0)


def _make_gather(j0, kc):
    mesh = plsc.VectorSubcoreMesh(core_axis_name="c", subcore_axis_name="s")

    @functools.partial(
        pl.kernel,
        mesh=mesh,
        compiler_params=pltpu.CompilerParams(use_tc_tiling_on_sc=False,
                                             needs_layout_passes=False),
        out_type=[
            jax.ShapeDtypeStruct((NW * kc, TW, CH), jnp.float32),
            jax.ShapeDtypeStruct((NW * kc, TW, CH), jnp.float32),
        ],
        scratch_types=[
            pltpu.VMEM((kc, CH), jnp.int32),
            pltpu.VMEM((kc, CH), jnp.int32),
            pltpu.VMEM((2, CH, TW), jnp.float32),
            pltpu.VMEM((2, CH, TW), jnp.float32),
            pltpu.VMEM((2, TW, CH), jnp.float32),
            pltpu.VMEM((2, TW, CH), jnp.float32),
            pltpu.SemaphoreType.DMA,
            pltpu.SemaphoreType.DMA,
            pltpu.SemaphoreType.DMA,
            pltpu.SemaphoreType.DMA,
        ],
    )
    def gather_k(tsrc, tdst, src2d, dst2d, gsrc, gdst,
                 idxs, idxd, rs, rd, ts, td, sg0, sg1, sw0, sw1):
        cid = lax.axis_index("c")
        sid = lax.axis_index("s")
        wid = sid * 2 + cid
        cbase = wid * kc

        pltpu.sync_copy(src2d.at[pl.ds(j0 + wid * kc, kc)], idxs)
        pltpu.sync_copy(dst2d.at[pl.ds(j0 + wid * kc, kc)], idxd)

        def start_gather(j, b):
            sem = sg0 if b == 0 else sg1
            pltpu.async_copy(tsrc.at[idxs.at[j]], rs.at[b], sem)
            pltpu.async_copy(tdst.at[idxd.at[j]], rd.at[b], sem)

        def wait_gather(b):
            sem = sg0 if b == 0 else sg1
            pltpu.make_async_copy(tsrc.at[idxs.at[0]], rs.at[b], sem).wait()
            pltpu.make_async_copy(tdst.at[idxd.at[0]], rd.at[b], sem).wait()

        def start_write(j, b):
            sem = sw0 if b == 0 else sw1
            pltpu.async_copy(ts.at[b], gsrc.at[cbase + j], sem)
            pltpu.async_copy(td.at[b], gdst.at[cbase + j], sem)

        def wait_write(b):
            sem = sw0 if b == 0 else sw1
            pltpu.make_async_copy(ts.at[b], gsrc.at[cbase], sem).wait()
            pltpu.make_async_copy(td.at[b], gdst.at[cbase], sem).wait()

        start_gather(0, 0)

        def body(j, carry):
            b = lax.rem(j, 2)

            @pl.when(jnp.logical_and(b == 0, j + 1 < kc))
            def _():
                start_gather(j + 1, 1)

            @pl.when(jnp.logical_and(b == 1, j + 1 < kc))
            def _():
                start_gather(j + 1, 0)

            @pl.when(b == 0)
            def _():
                wait_gather(0)

                @pl.when(j >= 2)
                def _():
                    wait_write(0)

            @pl.when(b == 1)
            def _():
                wait_gather(1)

                @pl.when(j >= 2)
                def _():
                    wait_write(1)

            _transpose_chunk(rs.at[b], ts.at[b])
            _transpose_chunk(rd.at[b], td.at[b])

            @pl.when(b == 0)
            def _():
                start_write(j, 0)

            @pl.when(b == 1)
            def _():
                start_write(j, 1)

            return carry

        lax.fori_loop(0, kc, body, 0)
        wait_write(0)
        wait_write(1)

    return gather_k


def _mlp_body(gs_ref, gd_ref, ed_ref, w1ct_ref, b1_ref, w2t_ref, b2_ref,
              w3t_ref, b3_ref, rept_ref, summt_ref, out_ref):
    gs = jnp.transpose(gs_ref[...], (1, 0, 2)).reshape(TW, EBLK)
    gd = jnp.transpose(gd_ref[...], (1, 0, 2)).reshape(TW, EBLK)
    asrc = gs[:16]
    ssrc = gs[16:]
    bdst = gd[:16]
    sdst = gd[16:]
    cpre = jnp.dot(w1ct_ref[...], ed_ref[...], preferred_element_type=jnp.float32)
    h1 = jax.nn.sigmoid(asrc + bdst + cpre + b1_ref[...])
    h2 = jax.nn.sigmoid(jnp.dot(w2t_ref[...], h1,
                                preferred_element_type=jnp.float32) + b2_ref[...])
    w = jax.nn.sigmoid(jnp.dot(w3t_ref[...], h2,
                               preferred_element_type=jnp.float32) + b3_ref[...])
    cat = jnp.concatenate([ssrc, sdst], axis=0)
    catx = jnp.dot(rept_ref[...], cat, preferred_element_type=jnp.float32)
    r = jnp.dot(summt_ref[...], catx * w, preferred_element_type=jnp.float32)
    alpha = jnp.where(r >= 0, r, 0.01 * r)
    ex = jnp.exp(alpha)
    exv = jnp.concatenate([ex, ex * ssrc], axis=0)
    out_ref[...] = jnp.transpose(exv.reshape(16, MB, CH), (1, 0, 2))


def _mlp(gsrc, gdst, edt, w1ct, b1, w2t, b2, w3t, b3, rept, summt, co=0):
    nch = gsrc.shape[0]
    grid = (nch // MB,)

    def full(shape):
        return pl.BlockSpec(shape, lambda i: tuple(0 for _ in shape))

    return pl.pallas_call(
        _mlp_body,
        grid=grid,
        in_specs=[
            pl.BlockSpec((MB, TW, CH), lambda i: (i, 0, 0)),
            pl.BlockSpec((MB, TW, CH), lambda i: (i, 0, 0)),
            pl.BlockSpec((DD, EBLK), lambda i: (0, i + co)),
            full((16, DD)),
            full((16, 1)),
            full((2, 16)),
            full((2, 1)),
            full((128, 2)),
            full((128, 1)),
            full((128, 16)),
            full((8, 128)),
        ],
        out_specs=pl.BlockSpec((MB, 16, CH), lambda i: (i, 0, 0)),
        out_shape=jax.ShapeDtypeStruct((nch, 16, CH), jnp.float32),
    )(gsrc, gdst, edt, w1ct, b1, w2t, b2, w3t, b3, rept, summt)


def _make_scatter(j0, kc):
    mesh = plsc.VectorSubcoreMesh(core_axis_name="c", subcore_axis_name="s")
    stripe = N_PAD // 16

    @functools.partial(
        pl.kernel,
        mesh=mesh,
        compiler_params=pltpu.CompilerParams(use_tc_tiling_on_sc=False,
                                             needs_layout_passes=False),
        out_type=jax.ShapeDtypeStruct((2, N_PAD, 16), jnp.float32),
        scratch_types=[
            pltpu.VMEM_SHARED((N_PAD, 16), jnp.float32),
            pltpu.VMEM((kc, CH), jnp.int32),
            pltpu.VMEM((2, 16, CH), jnp.float32),
            pltpu.VMEM((2, CH, 16), jnp.float32),
            pltpu.SemaphoreType.DMA,
            pltpu.SemaphoreType.DMA,
            pltpu.SemaphoreType.DMA,
            pltpu.SemaphoreType.DMA,
        ],
    )
    def scatter_k(exv, dst2d, zeros, partials, acc, idxd, slab, val,
                  sv0, sv1, sa0, sa1):
        cid = lax.axis_index("c")
        sid = lax.axis_index("s")
        wid = sid * 2 + cid
        cbase = wid * kc
        pltpu.sync_copy(zeros.at[pl.ds(sid * stripe, stripe)],
                        acc.at[pl.ds(sid * stripe, stripe)])
        pltpu.sync_copy(dst2d.at[pl.ds(j0 + wid * kc, kc)], idxd)
        plsc.subcore_barrier()

        def start_load(j, b):
            pltpu.async_copy(exv.at[cbase + j], slab.at[b], sv0 if b == 0 else sv1)

        def wait_load(b):
            pltpu.make_async_copy(exv.at[cbase], slab.at[b],
                                  sv0 if b == 0 else sv1).wait()

        def start_add(j, b):
            pltpu.async_copy(val.at[b], acc.at[idxd.at[j]], sa0 if b == 0 else sa1,
                             add=True)

        def wait_add(b):
            pltpu.make_async_copy(val.at[b], acc.at[idxd.at[0]],
                                  sa0 if b == 0 else sa1).wait()

        def untranspose(b):
            slab_b = slab.at[b]
            val_b = val.at[b]

            def group(g, carry):
                e0 = g * 16
                eidx = lax.iota(jnp.int32, 16) + e0
                for t in range(16):
                    v = slab_b[t, pl.ds(e0, 16)]
                    plsc.store_scatter(val_b, [eidx, jnp.full((16,), t, jnp.int32)], v)
                return carry

            lax.fori_loop(0, CH // 16, group, 0)

        start_load(0, 0)

        def body(j, carry):
            b = lax.rem(j, 2)

            @pl.when(jnp.logical_and(b == 0, j + 1 < kc))
            def _():
                start_load(j + 1, 1)

            @pl.when(jnp.logical_and(b == 1, j + 1 < kc))
            def _():
                start_load(j + 1, 0)

            @pl.when(b == 0)
            def _():
                wait_load(0)

                @pl.when(j >= 2)
                def _():
                    wait_add(0)

            @pl.when(b == 1)
            def _():
                wait_load(1)

                @pl.when(j >= 2)
                def _():
                    wait_add(1)

            untranspose(b)

            @pl.when(b == 0)
            def _():
                start_add(j, 0)

            @pl.when(b == 1)
            def _():
                start_add(j, 1)

            return carry

        lax.fori_loop(0, kc, body, 0)
        wait_add(0)
        wait_add(1)
        plsc.subcore_barrier()

        @pl.when(sid == 0)
        def _():
            pltpu.sync_copy(acc, partials.at[cid])

    return scatter_k


def _fin_body(p_ref, q_ref, r_ref, s_ref, ws_ref, out_ref):
    acc = p_ref[0] + p_ref[1] + q_ref[0] + q_ref[1] + r_ref[0] + r_ref[1] + s_ref[0] + s_ref[1]
    d = acc[:, :8]
    nu = acc[:, 8:]
    b = jax.nn.sigmoid(ws_ref[...])
    a = nu / (d + 1e-9)
    out_ref[...] = jnp.maximum(a * b, 0.0)


def _finalize(p0, p1, p2, p3, ws):
    grid = (N_PAD // NBLK,)
    return pl.pallas_call(
        _fin_body,
        grid=grid,
        in_specs=[
            pl.BlockSpec((2, NBLK, 16), lambda i: (0, i, 0)),
            pl.BlockSpec((2, NBLK, 16), lambda i: (0, i, 0)),
            pl.BlockSpec((2, NBLK, 16), lambda i: (0, i, 0)),
            pl.BlockSpec((2, NBLK, 16), lambda i: (0, i, 0)),
            pl.BlockSpec((1, 1), lambda i: (0, 0)),
        ],
        out_specs=pl.BlockSpec((NBLK, H), lambda i: (i, 0)),
        out_shape=jax.ShapeDtypeStruct((N_PAD, H), jnp.float32),
    )(p0, p1, p2, p3, ws)


def kernel(state, feature, edge_dist, W1, b1, W2, b2, W3, b3, w_scalar, src, dst):
    state_p = jnp.pad(state, ((0, N_PAD - N_NODES), (0, 0)))
    feature_p = jnp.pad(feature, ((0, N_PAD - N_NODES), (0, 0)))
    src_p = jnp.pad(src, (0, E_PAD - N_EDGES))
    dst_p = jnp.pad(dst, (0, E_PAD - N_EDGES), constant_values=N_NODES)
    edt = jnp.pad(edge_dist, ((0, E_PAD - N_EDGES), (0, 0))).T

    w1a = W1[:F]
    w1b = W1[F:2 * F]
    w1ct = W1[2 * F:].T
    rept = jnp.asarray(np.kron(np.eye(16, dtype=np.float32),
                               np.ones((8, 1), np.float32)))
    summt = jnp.asarray(np.tile(np.eye(8, dtype=np.float32), (1, 16)))
    zeros = jnp.zeros((N_PAD, 16), jnp.float32)
    src2d = src_p.reshape(NW * K, CH)
    dst2d = dst_p.reshape(NW * K, CH)

    tsrc, tdst = _prep(feature_p, state_p, w1a, w1b)
    hb = NW * KH
    wargs = (w1ct, b1.reshape(16, 1), W2.T, b2.reshape(2, 1),
             W3.T, b3.reshape(128, 1), rept, summt)
    gs = [_make_gather(q * hb, KH)(tsrc, tdst, src2d, dst2d) for q in range(4)]
    ps = []
    for q in range(4):
        exv = _mlp(gs[q][0], gs[q][1], edt, *wargs, co=q * hb // MB)
        ps.append(_make_scatter(q * hb, KH)(exv, dst2d, zeros))
    out_p = _finalize(ps[0], ps[1], ps[2], ps[3], w_scalar.reshape(1, 1))
    return out_p[:N_NODES]


# speedup vs baseline: 1.0717x; 1.0717x over previous
import functools

import jax
import jax.numpy as jnp
import numpy as np
from jax import lax
from jax.experimental import pallas as pl
from jax.experimental.pallas import tpu as pltpu
from jax.experimental.pallas import tpu_sc as plsc

N_NODES = 10000
N_EDGES = 320000
H = 8
F = 40
DD = 16

NW = 32
CH = 128
K = 80
KH = 40
EW = K * CH
E_PAD = NW * EW
NC = NW * K
N_PAD = 10240
NBLK = 256
TW = 24
MB = 32
EBLK = MB * CH


def _prep_body(f_ref, s_ref, w1a_ref, w1b_ref, tsrc_ref, tdst_ref):
    f = f_ref[...]
    s = s_ref[...]
    a = jnp.dot(f, w1a_ref[...], preferred_element_type=jnp.float32)
    b = jnp.dot(f, w1b_ref[...], preferred_element_type=jnp.float32)
    tsrc_ref[...] = jnp.concatenate([a, s], axis=1)
    tdst_ref[...] = jnp.concatenate([b, s], axis=1)


def _prep(feature_p, state_p, w1a, w1b):
    grid = (N_PAD // NBLK,)
    return pl.pallas_call(
        _prep_body,
        grid=grid,
        in_specs=[
            pl.BlockSpec((NBLK, F), lambda i: (i, 0)),
            pl.BlockSpec((NBLK, H), lambda i: (i, 0)),
            pl.BlockSpec((F, 16), lambda i: (0, 0)),
            pl.BlockSpec((F, 16), lambda i: (0, 0)),
        ],
        out_specs=[
            pl.BlockSpec((NBLK, TW), lambda i: (i, 0)),
            pl.BlockSpec((NBLK, TW), lambda i: (i, 0)),
        ],
        out_shape=[
            jax.ShapeDtypeStruct((N_PAD, TW), jnp.float32),
            jax.ShapeDtypeStruct((N_PAD, TW), jnp.float32),
        ],
    )(feature_p, state_p, w1a, w1b)


def _transpose_chunk(rows_b, slab_b):
    def group(g, carry):
        e0 = g * 16
        eidx = lax.iota(jnp.int32, 16) + e0
        for t in range(TW):
            v = plsc.load_gather(rows_b, [eidx, jnp.full((16,), t, jnp.int32)])
            slab_b[t, pl.ds(e0, 16)] = v
        return carry

    lax.fori_loop(0, CH // 16, group, 0)


def _make_gather(j0, kc):
    mesh = plsc.VectorSubcoreMesh(core_axis_name="c", subcore_axis_name="s")

    @functools.partial(
        pl.kernel,
        mesh=mesh,
        compiler_params=pltpu.CompilerParams(use_tc_tiling_on_sc=False,
                                             needs_layout_passes=False),
        out_type=[
            jax.ShapeDtypeStruct((NW * kc, TW, CH), jnp.float32),
            jax.ShapeDtypeStruct((NW * kc, TW, CH), jnp.float32),
        ],
        scratch_types=[
            pltpu.VMEM((kc, CH), jnp.int32),
            pltpu.VMEM((kc, CH), jnp.int32),
            pltpu.VMEM((2, CH, TW), jnp.float32),
            pltpu.VMEM((2, CH, TW), jnp.float32),
            pltpu.VMEM((2, TW, CH), jnp.float32),
            pltpu.VMEM((2, TW, CH), jnp.float32),
            pltpu.SemaphoreType.DMA,
            pltpu.SemaphoreType.DMA,
            pltpu.SemaphoreType.DMA,
            pltpu.SemaphoreType.DMA,
        ],
    )
    def gather_k(tsrc, tdst, src2d, dst2d, gsrc, gdst,
                 idxs, idxd, rs, rd, ts, td, sg0, sg1, sw0, sw1):
        cid = lax.axis_index("c")
        sid = lax.axis_index("s")
        wid = sid * 2 + cid
        cbase = wid * kc

        pltpu.sync_copy(src2d.at[pl.ds(j0 + wid * kc, kc)], idxs)
        pltpu.sync_copy(dst2d.at[pl.ds(j0 + wid * kc, kc)], idxd)

        def start_gather(j, b):
            sem = sg0 if b == 0 else sg1
            pltpu.async_copy(tsrc.at[idxs.at[j]], rs.at[b], sem)
            pltpu.async_copy(tdst.at[idxd.at[j]], rd.at[b], sem)

        def wait_gather(b):
            sem = sg0 if b == 0 else sg1
            pltpu.make_async_copy(tsrc.at[idxs.at[0]], rs.at[b], sem).wait()
            pltpu.make_async_copy(tdst.at[idxd.at[0]], rd.at[b], sem).wait()

        def start_write(j, b):
            sem = sw0 if b == 0 else sw1
            pltpu.async_copy(ts.at[b], gsrc.at[cbase + j], sem)
            pltpu.async_copy(td.at[b], gdst.at[cbase + j], sem)

        def wait_write(b):
            sem = sw0 if b == 0 else sw1
            pltpu.make_async_copy(ts.at[b], gsrc.at[cbase], sem).wait()
            pltpu.make_async_copy(td.at[b], gdst.at[cbase], sem).wait()

        start_gather(0, 0)

        def body(j, carry):
            b = lax.rem(j, 2)

            @pl.when(jnp.logical_and(b == 0, j + 1 < kc))
            def _():
                start_gather(j + 1, 1)

            @pl.when(jnp.logical_and(b == 1, j + 1 < kc))
            def _():
                start_gather(j + 1, 0)

            @pl.when(b == 0)
            def _():
                wait_gather(0)

                @pl.when(j >= 2)
                def _():
                    wait_write(0)

            @pl.when(b == 1)
            def _():
                wait_gather(1)

                @pl.when(j >= 2)
                def _():
                    wait_write(1)

            _transpose_chunk(rs.at[b], ts.at[b])
            _transpose_chunk(rd.at[b], td.at[b])

            @pl.when(b == 0)
            def _():
                start_write(j, 0)

            @pl.when(b == 1)
            def _():
                start_write(j, 1)

            return carry

        lax.fori_loop(0, kc, body, 0)
        wait_write(0)
        wait_write(1)

    return gather_k


def _mlp_body(gs_ref, gd_ref, ed_ref, w1ct_ref, b1_ref, w2t_ref, b2_ref,
              w3t_ref, b3_ref, rept_ref, summt_ref, out_ref):
    gs = jnp.transpose(gs_ref[...], (1, 0, 2)).reshape(TW, EBLK)
    gd = jnp.transpose(gd_ref[...], (1, 0, 2)).reshape(TW, EBLK)
    asrc = gs[:16]
    ssrc = gs[16:]
    bdst = gd[:16]
    sdst = gd[16:]
    cpre = jnp.dot(w1ct_ref[...], ed_ref[...], preferred_element_type=jnp.float32)
    h1 = jax.nn.sigmoid(asrc + bdst + cpre + b1_ref[...])
    h2 = jax.nn.sigmoid(jnp.dot(w2t_ref[...], h1,
                                preferred_element_type=jnp.float32) + b2_ref[...])
    w = jax.nn.sigmoid(jnp.dot(w3t_ref[...], h2,
                               preferred_element_type=jnp.float32) + b3_ref[...])
    cat = jnp.concatenate([ssrc, sdst], axis=0)
    catx = jnp.dot(rept_ref[...], cat, preferred_element_type=jnp.float32)
    r = jnp.dot(summt_ref[...], catx * w, preferred_element_type=jnp.float32)
    alpha = jnp.where(r >= 0, r, 0.01 * r)
    ex = jnp.exp(alpha)
    exv = jnp.concatenate([ex, ex * ssrc], axis=0)
    out_ref[...] = jnp.transpose(exv.reshape(16, MB, CH), (1, 0, 2))


def _mlp(gsrc, gdst, edt, w1ct, b1, w2t, b2, w3t, b3, rept, summt, co=0):
    nch = gsrc.shape[0]
    grid = (nch // MB,)

    def full(shape):
        return pl.BlockSpec(shape, lambda i: tuple(0 for _ in shape))

    return pl.pallas_call(
        _mlp_body,
        grid=grid,
        in_specs=[
            pl.BlockSpec((MB, TW, CH), lambda i: (i, 0, 0)),
            pl.BlockSpec((MB, TW, CH), lambda i: (i, 0, 0)),
            pl.BlockSpec((DD, EBLK), lambda i: (0, i + co)),
            full((16, DD)),
            full((16, 1)),
            full((2, 16)),
            full((2, 1)),
            full((128, 2)),
            full((128, 1)),
            full((128, 16)),
            full((8, 128)),
        ],
        out_specs=pl.BlockSpec((MB, 16, CH), lambda i: (i, 0, 0)),
        out_shape=jax.ShapeDtypeStruct((nch, 16, CH), jnp.float32),
    )(gsrc, gdst, edt, w1ct, b1, w2t, b2, w3t, b3, rept, summt)


def _make_scatter(j0, kc):
    mesh = plsc.VectorSubcoreMesh(core_axis_name="c", subcore_axis_name="s")
    stripe = N_PAD // 16

    @functools.partial(
        pl.kernel,
        mesh=mesh,
        compiler_params=pltpu.CompilerParams(use_tc_tiling_on_sc=False,
                                             needs_layout_passes=False),
        out_type=jax.ShapeDtypeStruct((2, N_PAD, 16), jnp.float32),
        scratch_types=[
            pltpu.VMEM_SHARED((N_PAD, 16), jnp.float32),
            pltpu.VMEM((kc, CH), jnp.int32),
            pltpu.VMEM((2, 16, CH), jnp.float32),
            pltpu.VMEM((2, CH, 16), jnp.float32),
            pltpu.SemaphoreType.DMA,
            pltpu.SemaphoreType.DMA,
            pltpu.SemaphoreType.DMA,
            pltpu.SemaphoreType.DMA,
        ],
    )
    def scatter_k(exv, dst2d, zeros, partials, acc, idxd, slab, val,
                  sv0, sv1, sa0, sa1):
        cid = lax.axis_index("c")
        sid = lax.axis_index("s")
        wid = sid * 2 + cid
        cbase = wid * kc
        pltpu.sync_copy(zeros.at[pl.ds(sid * stripe, stripe)],
                        acc.at[pl.ds(sid * stripe, stripe)])
        pltpu.sync_copy(dst2d.at[pl.ds(j0 + wid * kc, kc)], idxd)
        plsc.subcore_barrier()

        def start_load(j, b):
            pltpu.async_copy(exv.at[cbase + j], slab.at[b], sv0 if b == 0 else sv1)

        def wait_load(b):
            pltpu.make_async_copy(exv.at[cbase], slab.at[b],
                                  sv0 if b == 0 else sv1).wait()

        def start_add(j, b):
            pltpu.async_copy(val.at[b], acc.at[idxd.at[j]], sa0 if b == 0 else sa1,
                             add=True)

        def wait_add(b):
            pltpu.make_async_copy(val.at[b], acc.at[idxd.at[0]],
                                  sa0 if b == 0 else sa1).wait()

        def untranspose(b):
            slab_b = slab.at[b]
            val_b = val.at[b]

            def group(g, carry):
                e0 = g * 16
                eidx = lax.iota(jnp.int32, 16) + e0
                for t in range(16):
                    v = slab_b[t, pl.ds(e0, 16)]
                    plsc.store_scatter(val_b, [eidx, jnp.full((16,), t, jnp.int32)], v)
                return carry

            lax.fori_loop(0, CH // 16, group, 0)

        start_load(0, 0)

        def body(j, carry):
            b = lax.rem(j, 2)

            @pl.when(jnp.logical_and(b == 0, j + 1 < kc))
            def _():
                start_load(j + 1, 1)

            @pl.when(jnp.logical_and(b == 1, j + 1 < kc))
            def _():
                start_load(j + 1, 0)

            @pl.when(b == 0)
            def _():
                wait_load(0)

                @pl.when(j >= 2)
                def _():
                    wait_add(0)

            @pl.when(b == 1)
            def _():
                wait_load(1)

                @pl.when(j >= 2)
                def _():
                    wait_add(1)

            untranspose(b)

            @pl.when(b == 0)
            def _():
                start_add(j, 0)

            @pl.when(b == 1)
            def _():
                start_add(j, 1)

            return carry

        lax.fori_loop(0, kc, body, 0)
        wait_add(0)
        wait_add(1)
        plsc.subcore_barrier()

        @pl.when(sid == 0)
        def _():
            pltpu.sync_copy(acc, partials.at[cid])

    return scatter_k


def _fin_body(p_ref, q_ref, ws_ref, out_ref):
    acc = p_ref[0] + p_ref[1] + q_ref[0] + q_ref[1]
    d = acc[:, :8]
    nu = acc[:, 8:]
    b = jax.nn.sigmoid(ws_ref[...])
    a = nu / (d + 1e-9)
    out_ref[...] = jnp.maximum(a * b, 0.0)


def _finalize(p0, p1, ws):
    grid = (N_PAD // NBLK,)
    return pl.pallas_call(
        _fin_body,
        grid=grid,
        in_specs=[
            pl.BlockSpec((2, NBLK, 16), lambda i: (0, i, 0)),
            pl.BlockSpec((2, NBLK, 16), lambda i: (0, i, 0)),
            pl.BlockSpec((1, 1), lambda i: (0, 0)),
        ],
        out_specs=pl.BlockSpec((NBLK, H), lambda i: (i, 0)),
        out_shape=jax.ShapeDtypeStruct((N_PAD, H), jnp.float32),
    )(p0, p1, ws)


def kernel(state, feature, edge_dist, W1, b1, W2, b2, W3, b3, w_scalar, src, dst):
    state_p = jnp.pad(state, ((0, N_PAD - N_NODES), (0, 0)))
    feature_p = jnp.pad(feature, ((0, N_PAD - N_NODES), (0, 0)))
    src_p = jnp.pad(src, (0, E_PAD - N_EDGES))
    dst_p = jnp.pad(dst, (0, E_PAD - N_EDGES), constant_values=N_NODES)
    edt = jnp.pad(edge_dist, ((0, E_PAD - N_EDGES), (0, 0))).T

    w1a = W1[:F]
    w1b = W1[F:2 * F]
    w1ct = W1[2 * F:].T
    rept = jnp.asarray(np.kron(np.eye(16, dtype=np.float32),
                               np.ones((8, 1), np.float32)))
    summt = jnp.asarray(np.tile(np.eye(8, dtype=np.float32), (1, 16)))
    zeros = jnp.zeros((N_PAD, 16), jnp.float32)
    src2d = src_p.reshape(NW * K, CH)
    dst2d = dst_p.reshape(NW * K, CH)

    tsrc, tdst = _prep(feature_p, state_p, w1a, w1b)
    hb = NW * KH
    wargs = (w1ct, b1.reshape(16, 1), W2.T, b2.reshape(2, 1),
             W3.T, b3.reshape(128, 1), rept, summt)
    gs = [_make_gather(q * hb, KH)(tsrc, tdst, src2d, dst2d) for q in range(2)]
    ps = []
    for q in range(2):
        exv = _mlp(gs[q][0], gs[q][1], edt, *wargs, co=q * hb // MB)
        ps.append(_make_scatter(q * hb, KH)(exv, dst2d, zeros))
    out_p = _finalize(ps[0], ps[1], w_scalar.reshape(1, 1))
    return out_p[:N_NODES]


# speedup vs baseline: 1.0900x; 1.0171x over previous
import functools

import jax
import jax.numpy as jnp
import numpy as np
from jax import lax
from jax.experimental import pallas as pl
from jax.experimental.pallas import tpu as pltpu
from jax.experimental.pallas import tpu_sc as plsc

N_NODES = 10000
N_EDGES = 320000
H = 8
F = 40
DD = 16

NW = 32
CH = 128
K = 80
KH = 40
EW = K * CH
E_PAD = NW * EW
NC = NW * K
N_PAD = 10240
NBLK = 256
TW = 24
MB = 64
EBLK = MB * CH


def _prep_body(f_ref, s_ref, w1a_ref, w1b_ref, tsrc_ref, tdst_ref):
    f = f_ref[...]
    s = s_ref[...]
    a = jnp.dot(f, w1a_ref[...], preferred_element_type=jnp.float32)
    b = jnp.dot(f, w1b_ref[...], preferred_element_type=jnp.float32)
    tsrc_ref[...] = jnp.concatenate([a, s], axis=1)
    tdst_ref[...] = jnp.concatenate([b, s], axis=1)


def _prep(feature_p, state_p, w1a, w1b):
    grid = (N_PAD // NBLK,)
    return pl.pallas_call(
        _prep_body,
        grid=grid,
        in_specs=[
            pl.BlockSpec((NBLK, F), lambda i: (i, 0)),
            pl.BlockSpec((NBLK, H), lambda i: (i, 0)),
            pl.BlockSpec((F, 16), lambda i: (0, 0)),
            pl.BlockSpec((F, 16), lambda i: (0, 0)),
        ],
        out_specs=[
            pl.BlockSpec((NBLK, TW), lambda i: (i, 0)),
            pl.BlockSpec((NBLK, TW), lambda i: (i, 0)),
        ],
        out_shape=[
            jax.ShapeDtypeStruct((N_PAD, TW), jnp.float32),
            jax.ShapeDtypeStruct((N_PAD, TW), jnp.float32),
        ],
    )(feature_p, state_p, w1a, w1b)


def _transpose_chunk(rows_b, slab_b):
    def group(g, carry):
        e0 = g * 16
        eidx = lax.iota(jnp.int32, 16) + e0
        for t in range(TW):
            v = plsc.load_gather(rows_b, [eidx, jnp.full((16,), t, jnp.int32)])
            slab_b[t, pl.ds(e0, 16)] = v
        return carry

    lax.fori_loop(0, CH // 16, group, 0)


def _make_gather(j0, kc):
    mesh = plsc.VectorSubcoreMesh(core_axis_name="c", subcore_axis_name="s")

    @functools.partial(
        pl.kernel,
        mesh=mesh,
        compiler_params=pltpu.CompilerParams(use_tc_tiling_on_sc=False,
                                             needs_layout_passes=False),
        out_type=[
            jax.ShapeDtypeStruct((NW * kc, TW, CH), jnp.float32),
            jax.ShapeDtypeStruct((NW * kc, TW, CH), jnp.float32),
        ],
        scratch_types=[
            pltpu.VMEM((kc, CH), jnp.int32),
            pltpu.VMEM((kc, CH), jnp.int32),
            pltpu.VMEM((2, CH, TW), jnp.float32),
            pltpu.VMEM((2, CH, TW), jnp.float32),
            pltpu.VMEM((2, TW, CH), jnp.float32),
            pltpu.VMEM((2, TW, CH), jnp.float32),
            pltpu.SemaphoreType.DMA,
            pltpu.SemaphoreType.DMA,
            pltpu.SemaphoreType.DMA,
            pltpu.SemaphoreType.DMA,
        ],
    )
    def gather_k(tsrc, tdst, src2d, dst2d, gsrc, gdst,
                 idxs, idxd, rs, rd, ts, td, sg0, sg1, sw0, sw1):
        cid = lax.axis_index("c")
        sid = lax.axis_index("s")
        wid = sid * 2 + cid
        cbase = wid * kc

        pltpu.sync_copy(src2d.at[pl.ds(j0 + wid * kc, kc)], idxs)
        pltpu.sync_copy(dst2d.at[pl.ds(j0 + wid * kc, kc)], idxd)

        def start_gather(j, b):
            sem = sg0 if b == 0 else sg1
            pltpu.async_copy(tsrc.at[idxs.at[j]], rs.at[b], sem)
            pltpu.async_copy(tdst.at[idxd.at[j]], rd.at[b], sem)

        def wait_gather(b):
            sem = sg0 if b == 0 else sg1
            pltpu.make_async_copy(tsrc.at[idxs.at[0]], rs.at[b], sem).wait()
            pltpu.make_async_copy(tdst.at[idxd.at[0]], rd.at[b], sem).wait()

        def start_write(j, b):
            sem = sw0 if b == 0 else sw1
            pltpu.async_copy(ts.at[b], gsrc.at[cbase + j], sem)
            pltpu.async_copy(td.at[b], gdst.at[cbase + j], sem)

        def wait_write(b):
            sem = sw0 if b == 0 else sw1
            pltpu.make_async_copy(ts.at[b], gsrc.at[cbase], sem).wait()
            pltpu.make_async_copy(td.at[b], gdst.at[cbase], sem).wait()

        start_gather(0, 0)

        def body(j, carry):
            b = lax.rem(j, 2)

            @pl.when(jnp.logical_and(b == 0, j + 1 < kc))
            def _():
                start_gather(j + 1, 1)

            @pl.when(jnp.logical_and(b == 1, j + 1 < kc))
            def _():
                start_gather(j + 1, 0)

            @pl.when(b == 0)
            def _():
                wait_gather(0)

                @pl.when(j >= 2)
                def _():
                    wait_write(0)

            @pl.when(b == 1)
            def _():
                wait_gather(1)

                @pl.when(j >= 2)
                def _():
                    wait_write(1)

            _transpose_chunk(rs.at[b], ts.at[b])
            _transpose_chunk(rd.at[b], td.at[b])

            @pl.when(b == 0)
            def _():
                start_write(j, 0)

            @pl.when(b == 1)
            def _():
                start_write(j, 1)

            return carry

        lax.fori_loop(0, kc, body, 0)
        wait_write(0)
        wait_write(1)

    return gather_k


def _mlp_body(gs_ref, gd_ref, ed_ref, w1ct_ref, b1_ref, w2t_ref, b2_ref,
              w3t_ref, b3_ref, rept_ref, summt_ref, out_ref):
    gs = jnp.transpose(gs_ref[...], (1, 0, 2)).reshape(TW, EBLK)
    gd = jnp.transpose(gd_ref[...], (1, 0, 2)).reshape(TW, EBLK)
    asrc = gs[:16]
    ssrc = gs[16:]
    bdst = gd[:16]
    sdst = gd[16:]
    cpre = jnp.dot(w1ct_ref[...], ed_ref[...], preferred_element_type=jnp.float32)
    h1 = jax.nn.sigmoid(asrc + bdst + cpre + b1_ref[...])
    h2 = jax.nn.sigmoid(jnp.dot(w2t_ref[...], h1,
                                preferred_element_type=jnp.float32) + b2_ref[...])
    w = jax.nn.sigmoid(jnp.dot(w3t_ref[...], h2,
                               preferred_element_type=jnp.float32) + b3_ref[...])
    cat = jnp.concatenate([ssrc, sdst], axis=0)
    catx = jnp.dot(rept_ref[...], cat, preferred_element_type=jnp.float32)
    r = jnp.dot(summt_ref[...], catx * w, preferred_element_type=jnp.float32)
    alpha = jnp.where(r >= 0, r, 0.01 * r)
    ex = jnp.exp(alpha)
    exv = jnp.concatenate([ex, ex * ssrc], axis=0)
    out_ref[...] = jnp.transpose(exv.reshape(16, MB, CH), (1, 0, 2))


def _mlp(gsrc, gdst, edt, w1ct, b1, w2t, b2, w3t, b3, rept, summt, co=0):
    nch = gsrc.shape[0]
    grid = (nch // MB,)

    def full(shape):
        return pl.BlockSpec(shape, lambda i: tuple(0 for _ in shape))

    return pl.pallas_call(
        _mlp_body,
        grid=grid,
        in_specs=[
            pl.BlockSpec((MB, TW, CH), lambda i: (i, 0, 0)),
            pl.BlockSpec((MB, TW, CH), lambda i: (i, 0, 0)),
            pl.BlockSpec((DD, EBLK), lambda i: (0, i + co)),
            full((16, DD)),
            full((16, 1)),
            full((2, 16)),
            full((2, 1)),
            full((128, 2)),
            full((128, 1)),
            full((128, 16)),
            full((8, 128)),
        ],
        out_specs=pl.BlockSpec((MB, 16, CH), lambda i: (i, 0, 0)),
        out_shape=jax.ShapeDtypeStruct((nch, 16, CH), jnp.float32),
    )(gsrc, gdst, edt, w1ct, b1, w2t, b2, w3t, b3, rept, summt)


def _make_scatter(j0, kc):
    mesh = plsc.VectorSubcoreMesh(core_axis_name="c", subcore_axis_name="s")
    stripe = N_PAD // 16

    @functools.partial(
        pl.kernel,
        mesh=mesh,
        compiler_params=pltpu.CompilerParams(use_tc_tiling_on_sc=False,
                                             needs_layout_passes=False),
        out_type=jax.ShapeDtypeStruct((2, N_PAD, 16), jnp.float32),
        scratch_types=[
            pltpu.VMEM_SHARED((N_PAD, 16), jnp.float32),
            pltpu.VMEM((kc, CH), jnp.int32),
            pltpu.VMEM((2, 16, CH), jnp.float32),
            pltpu.VMEM((2, CH, 16), jnp.float32),
            pltpu.SemaphoreType.DMA,
            pltpu.SemaphoreType.DMA,
            pltpu.SemaphoreType.DMA,
            pltpu.SemaphoreType.DMA,
        ],
    )
    def scatter_k(exv, dst2d, zeros, partials, acc, idxd, slab, val,
                  sv0, sv1, sa0, sa1):
        cid = lax.axis_index("c")
        sid = lax.axis_index("s")
        wid = sid * 2 + cid
        cbase = wid * kc
        pltpu.sync_copy(zeros.at[pl.ds(sid * stripe, stripe)],
                        acc.at[pl.ds(sid * stripe, stripe)])
        pltpu.sync_copy(dst2d.at[pl.ds(j0 + wid * kc, kc)], idxd)
        plsc.subcore_barrier()

        def start_load(j, b):
            pltpu.async_copy(exv.at[cbase + j], slab.at[b], sv0 if b == 0 else sv1)

        def wait_load(b):
            pltpu.make_async_copy(exv.at[cbase], slab.at[b],
                                  sv0 if b == 0 else sv1).wait()

        def start_add(j, b):
            pltpu.async_copy(val.at[b], acc.at[idxd.at[j]], sa0 if b == 0 else sa1,
                             add=True)

        def wait_add(b):
            pltpu.make_async_copy(val.at[b], acc.at[idxd.at[0]],
                                  sa0 if b == 0 else sa1).wait()

        def untranspose(b):
            slab_b = slab.at[b]
            val_b = val.at[b]

            def group(g, carry):
                e0 = g * 16
                eidx = lax.iota(jnp.int32, 16) + e0
                for t in range(16):
                    v = slab_b[t, pl.ds(e0, 16)]
                    plsc.store_scatter(val_b, [eidx, jnp.full((16,), t, jnp.int32)], v)
                return carry

            lax.fori_loop(0, CH // 16, group, 0)

        start_load(0, 0)

        def body(j, carry):
            b = lax.rem(j, 2)

            @pl.when(jnp.logical_and(b == 0, j + 1 < kc))
            def _():
                start_load(j + 1, 1)

            @pl.when(jnp.logical_and(b == 1, j + 1 < kc))
            def _():
                start_load(j + 1, 0)

            @pl.when(b == 0)
            def _():
                wait_load(0)

                @pl.when(j >= 2)
                def _():
                    wait_add(0)

            @pl.when(b == 1)
            def _():
                wait_load(1)

                @pl.when(j >= 2)
                def _():
                    wait_add(1)

            untranspose(b)

            @pl.when(b == 0)
            def _():
                start_add(j, 0)

            @pl.when(b == 1)
            def _():
                start_add(j, 1)

            return carry

        lax.fori_loop(0, kc, body, 0)
        wait_add(0)
        wait_add(1)
        plsc.subcore_barrier()

        @pl.when(sid == 0)
        def _():
            pltpu.sync_copy(acc, partials.at[cid])

    return scatter_k


def _fin_body(p_ref, q_ref, ws_ref, out_ref):
    acc = p_ref[0] + p_ref[1] + q_ref[0] + q_ref[1]
    d = acc[:, :8]
    nu = acc[:, 8:]
    b = jax.nn.sigmoid(ws_ref[...])
    a = nu / (d + 1e-9)
    out_ref[...] = jnp.maximum(a * b, 0.0)


def _finalize(p0, p1, ws):
    grid = (N_PAD // NBLK,)
    return pl.pallas_call(
        _fin_body,
        grid=grid,
        in_specs=[
            pl.BlockSpec((2, NBLK, 16), lambda i: (0, i, 0)),
            pl.BlockSpec((2, NBLK, 16), lambda i: (0, i, 0)),
            pl.BlockSpec((1, 1), lambda i: (0, 0)),
        ],
        out_specs=pl.BlockSpec((NBLK, H), lambda i: (i, 0)),
        out_shape=jax.ShapeDtypeStruct((N_PAD, H), jnp.float32),
    )(p0, p1, ws)


def kernel(state, feature, edge_dist, W1, b1, W2, b2, W3, b3, w_scalar, src, dst):
    state_p = jnp.pad(state, ((0, N_PAD - N_NODES), (0, 0)))
    feature_p = jnp.pad(feature, ((0, N_PAD - N_NODES), (0, 0)))
    src_p = jnp.pad(src, (0, E_PAD - N_EDGES))
    dst_p = jnp.pad(dst, (0, E_PAD - N_EDGES), constant_values=N_NODES)
    edt = jnp.pad(edge_dist, ((0, E_PAD - N_EDGES), (0, 0))).T

    w1a = W1[:F]
    w1b = W1[F:2 * F]
    w1ct = W1[2 * F:].T
    rept = jnp.asarray(np.kron(np.eye(16, dtype=np.float32),
                               np.ones((8, 1), np.float32)))
    summt = jnp.asarray(np.tile(np.eye(8, dtype=np.float32), (1, 16)))
    zeros = jnp.zeros((N_PAD, 16), jnp.float32)
    src2d = src_p.reshape(NW * K, CH)
    dst2d = dst_p.reshape(NW * K, CH)

    tsrc, tdst = _prep(feature_p, state_p, w1a, w1b)
    hb = NW * KH
    wargs = (w1ct, b1.reshape(16, 1), W2.T, b2.reshape(2, 1),
             W3.T, b3.reshape(128, 1), rept, summt)
    gs = [_make_gather(q * hb, KH)(tsrc, tdst, src2d, dst2d) for q in range(2)]
    ps = []
    for q in range(2):
        exv = _mlp(gs[q][0], gs[q][1], edt, *wargs, co=q * hb // MB)
        ps.append(_make_scatter(q * hb, KH)(exv, dst2d, zeros))
    out_p = _finalize(ps[0], ps[1], w_scalar.reshape(1, 1))
    return out_p[:N_NODES]
